# baseline (speedup 1.0000x reference)
import jax
import jax.numpy as jnp
from jax import lax
from jax.experimental import pallas as pl
from jax.experimental.pallas import tpu as pltpu
from jax.experimental.pallas import tpu_sc as plsc

_NC = 2
_NS = 16
_NW = _NC * _NS

_CHUNK = 128


def _emb_body(n_chunks, per_w, d, table_hbm, idx_hbm, out_hbm,
              idx_v, rows_v, sem):
    wid = lax.axis_index("s") * _NC + lax.axis_index("c")
    base_w = wid * per_w

    def body(i, carry):
        base = base_w + i * _CHUNK
        pltpu.sync_copy(idx_hbm.at[pl.ds(base, _CHUNK)], idx_v)
        pltpu.async_copy(table_hbm.at[idx_v], rows_v, sem).wait()
        pltpu.sync_copy(rows_v, out_hbm.at[pl.ds(base, _CHUNK)])
        return carry

    lax.fori_loop(0, n_chunks, body, 0)


def kernel(input, weight):
    b, h = input.shape
    v, d = weight.shape
    n = b * h
    per_w = n // _NW
    n_chunks = per_w // _CHUNK
    assert per_w * _NW == n and n_chunks * _CHUNK == per_w

    idx = input.reshape(-1).astype(jnp.int32)
    mesh = plsc.VectorSubcoreMesh(core_axis_name="c", subcore_axis_name="s")

    def body(table_hbm, idx_hbm, out_hbm, idx_v, rows_v, sem):
        _emb_body(n_chunks, per_w, d, table_hbm, idx_hbm, out_hbm,
                  idx_v, rows_v, sem)

    out = pl.kernel(
        body,
        mesh=mesh,
        compiler_params=pltpu.CompilerParams(use_tc_tiling_on_sc=False),
        out_type=jax.ShapeDtypeStruct((n, d), jnp.float32),
        scratch_types=[
            pltpu.VMEM((_CHUNK,), jnp.int32),
            pltpu.VMEM((_CHUNK, d), jnp.float32),
            pltpu.SemaphoreType.DMA,
        ],
    )(weight, idx)
    return out.reshape(b, h, d)


# speedup vs baseline: 1.0197x; 1.0197x over previous
import jax
import jax.numpy as jnp
from jax import lax
from jax.experimental import pallas as pl
from jax.experimental.pallas import tpu as pltpu
from jax.experimental.pallas import tpu_sc as plsc

_NC = 2
_NS = 16
_NW = _NC * _NS

_CHUNK = 128
_NBUF = 3


def kernel(input, weight):
    b, h = input.shape
    v, d = weight.shape
    n = b * h
    per_w = n // _NW
    n_chunks = per_w // _CHUNK
    assert per_w * _NW == n and n_chunks * _CHUNK == per_w

    idx = input.reshape(-1).astype(jnp.int32)
    mesh = plsc.VectorSubcoreMesh(core_axis_name="c", subcore_axis_name="s")

    def body(table_hbm, idx_hbm, out_hbm,
             i0, i1, i2, r0, r1, r2,
             gi0, gi1, gi2, gg0, gg1, gg2, gs0, gs1, gs2):
        ibuf = (i0, i1, i2)
        rows = (r0, r1, r2)
        isem = (gi0, gi1, gi2)
        gsem = (gg0, gg1, gg2)
        ssem = (gs0, gs1, gs2)
        wid = lax.axis_index("s") * _NC + lax.axis_index("c")
        base_w = wid * per_w

        def start_ic(i):
            bi = i % _NBUF
            return pltpu.async_copy(
                idx_hbm.at[pl.ds(base_w + i * _CHUNK, _CHUNK)],
                ibuf[bi], isem[bi])

        def start_g(i):
            bi = i % _NBUF
            return pltpu.async_copy(table_hbm.at[ibuf[bi]], rows[bi],
                                    gsem[bi])

        def start_s(i):
            bi = i % _NBUF
            return pltpu.async_copy(
                rows[bi], out_hbm.at[pl.ds(base_w + i * _CHUNK, _CHUNK)],
                ssem[bi])

        nb = min(_NBUF, n_chunks)
        ic = [None] * n_chunks
        g = [None] * n_chunks
        s = [None] * n_chunks
        for i in range(nb):
            ic[i] = start_ic(i)
        ic[0].wait()
        g[0] = start_g(0)
        for i in range(n_chunks):
            if i + 1 < n_chunks:
                if i + 1 - _NBUF >= 0:
                    s[i + 1 - _NBUF].wait()
                ic[i + 1].wait()
                g[i + 1] = start_g(i + 1)
            g[i].wait()
            s[i] = start_s(i)
            if i + _NBUF < n_chunks:
                ic[i + _NBUF] = start_ic(i + _NBUF)
        for i in range(max(0, n_chunks - _NBUF), n_chunks):
            s[i].wait()

    out = pl.kernel(
        body,
        mesh=mesh,
        compiler_params=pltpu.CompilerParams(use_tc_tiling_on_sc=False),
        out_type=jax.ShapeDtypeStruct((n, d), jnp.float32),
        scratch_types=[
            pltpu.VMEM((_CHUNK,), jnp.int32),
            pltpu.VMEM((_CHUNK,), jnp.int32),
            pltpu.VMEM((_CHUNK,), jnp.int32),
            pltpu.VMEM((_CHUNK, d), jnp.float32),
            pltpu.VMEM((_CHUNK, d), jnp.float32),
            pltpu.VMEM((_CHUNK, d), jnp.float32),
            pltpu.SemaphoreType.DMA,
            pltpu.SemaphoreType.DMA,
            pltpu.SemaphoreType.DMA,
            pltpu.SemaphoreType.DMA,
            pltpu.SemaphoreType.DMA,
            pltpu.SemaphoreType.DMA,
            pltpu.SemaphoreType.DMA,
            pltpu.SemaphoreType.DMA,
            pltpu.SemaphoreType.DMA,
        ],
    )(weight, idx)
    return out.reshape(b, h, d)


# speedup vs baseline: 4.6560x; 4.5661x over previous
import jax
import jax.numpy as jnp
from jax import lax
from jax.experimental import pallas as pl
from jax.experimental.pallas import tpu as pltpu
from jax.experimental.pallas import tpu_sc as plsc

_NC = 2
_NS = 16
_NW = _NC * _NS

_BCHUNK = 8
_L = 16


def kernel(input, weight):
    b, h = input.shape
    v, d = weight.shape
    n = b * h
    b_per_w = b // _NW
    n_chunks = b_per_w // _BCHUNK
    rows_per_chunk = _BCHUNK * h
    assert b_per_w * _NW == b and n_chunks * _BCHUNK == b_per_w

    idx = input.reshape(-1).astype(jnp.int32)
    mesh = plsc.VectorSubcoreMesh(core_axis_name="c", subcore_axis_name="s")

    def body(tbl_hbm, idx_hbm, out_hbm, idx_v, rows_v, gsem):
        wid = lax.axis_index("s") * _NC + lax.axis_index("c")

        def chunk(i, carry):
            bbase = wid * b_per_w + i * _BCHUNK
            rbase = bbase * h
            pltpu.sync_copy(idx_hbm.at[pl.ds(rbase, rows_per_chunk)], idx_v)
            cps = []
            for g in range(rows_per_chunk // _L):
                vv = idx_v[pl.ds(g * _L, _L)]
                for l in range(_L):
                    r = g * _L + l
                    rowid = vv[l]
                    cps.append(pltpu.async_copy(
                        tbl_hbm.at[pl.ds(rowid, 1)],
                        rows_v.at[r // h, pl.ds(r % h, 1)], gsem))
            for cp in cps:
                cp.wait()
            pltpu.sync_copy(rows_v, out_hbm.at[pl.ds(bbase, _BCHUNK)])
            return carry

        lax.fori_loop(0, n_chunks, chunk, 0)

    out = pl.kernel(
        body,
        mesh=mesh,
        out_type=jax.ShapeDtypeStruct((b, h, d), jnp.float32),
        scratch_types=[
            pltpu.VMEM((rows_per_chunk,), jnp.int32),
            pltpu.VMEM((_BCHUNK, h, d), jnp.float32),
            pltpu.SemaphoreType.DMA,
        ],
    )(weight, idx)
    return out


# speedup vs baseline: 4.7600x; 1.0223x over previous
import jax
import jax.numpy as jnp
from jax import lax
from jax.experimental import pallas as pl
from jax.experimental.pallas import tpu as pltpu
from jax.experimental.pallas import tpu_sc as plsc

_NC = 2
_NS = 16
_NW = _NC * _NS

_BCHUNK = 8
_L = 16


def kernel(input, weight):
    b, h = input.shape
    v, d = weight.shape
    n = b * h
    b_per_w = b // _NW
    per_w = b_per_w * h
    n_chunks = b_per_w // _BCHUNK
    n_pairs = n_chunks // 2
    rpc = _BCHUNK * h
    assert b_per_w * _NW == b and n_pairs * 2 == n_chunks and rpc % _L == 0

    idx = input.reshape(-1).astype(jnp.int32)
    mesh = plsc.VectorSubcoreMesh(core_axis_name="c", subcore_axis_name="s")

    def body(tbl_hbm, idx_hbm, out_hbm, idx_v, rA, rB, gA, gB, sA, sB):
        wid = lax.axis_index("s") * _NC + lax.axis_index("c")
        wrow = wid * per_w
        wbatch = wid * b_per_w
        pltpu.sync_copy(idx_hbm.at[pl.ds(wrow, per_w)], idx_v)

        def issue(c, rbuf, gsem):
            cps = []
            for g in range(rpc // _L):
                vv = idx_v[pl.ds(c * rpc + g * _L, _L)]
                for l in range(_L):
                    r = g * _L + l
                    cps.append(pltpu.async_copy(
                        tbl_hbm.at[pl.ds(vv[l], 1)],
                        rbuf.at[r // h, pl.ds(r % h, 1)], gsem))
            return cps

        def store(c, rbuf, ssem):
            return pltpu.async_copy(
                rbuf, out_hbm.at[pl.ds(wbatch + c * _BCHUNK, _BCHUNK)], ssem)

        def wait_store(rbuf, ssem):
            pltpu.make_async_copy(
                rbuf, out_hbm.at[pl.ds(wbatch, _BCHUNK)], ssem).wait()

        def pair(j, carry):
            ca = 2 * j
            cb = 2 * j + 1

            @pl.when(j > 0)
            def _():
                wait_store(rA, sA)
            cpsA = issue(ca, rA, gA)

            @pl.when(j > 0)
            def _():
                wait_store(rB, sB)
            cpsB = issue(cb, rB, gB)

            for cp in cpsA:
                cp.wait()
            store(ca, rA, sA)
            for cp in cpsB:
                cp.wait()
            store(cb, rB, sB)
            return carry

        lax.fori_loop(0, n_pairs, pair, 0)
        wait_store(rA, sA)
        wait_store(rB, sB)

    out = pl.kernel(
        body,
        mesh=mesh,
        out_type=jax.ShapeDtypeStruct((b, h, d), jnp.float32),
        scratch_types=[
            pltpu.VMEM((per_w,), jnp.int32),
            pltpu.VMEM((_BCHUNK, h, d), jnp.float32),
            pltpu.VMEM((_BCHUNK, h, d), jnp.float32),
            pltpu.SemaphoreType.DMA,
            pltpu.SemaphoreType.DMA,
            pltpu.SemaphoreType.DMA,
            pltpu.SemaphoreType.DMA,
        ],
    )(weight, idx)
    return out
